# Initial kernel scaffold; baseline (speedup 1.0000x reference)
import jax
import jax.numpy as jnp
from jax.experimental import pallas as pl


def kernel(x, edge_index, W1, b1, W2, b2, Wl, bl):
    raise NotImplementedError("write your pallas kernel here")


# speedup vs baseline: 25.5708x; 25.5708x over previous
import functools

import jax
import jax.numpy as jnp
from jax import lax
from jax.experimental import pallas as pl
from jax.experimental.pallas import tpu as pltpu
from jax.experimental.pallas import tpu_sc as plsc

N_NODES = 10000
N_PAD = 10240
N_EDGES = 320000
E_PAD = 327680
BLK = 128
NC = 2
NS = 16
ROWS_PER_TILE = N_PAD // NS
NBLK_W = E_PAD // (NC * NS * BLK)
NBLK_S = E_PAD // (NS * BLK)

_mesh = plsc.VectorSubcoreMesh(
    core_axis_name="c", subcore_axis_name="s", num_cores=NC, num_subcores=NS
)
_sc_params = pltpu.CompilerParams(use_tc_tiling_on_sc=False)


@functools.partial(
    pl.kernel,
    out_type=jax.ShapeDtypeStruct((N_PAD,), jnp.float32),
    mesh=_mesh,
    scratch_types=[
        pltpu.VMEM((NBLK_S, BLK), jnp.int32),
        pltpu.VMEM((BLK,), jnp.float32),
        pltpu.VMEM((ROWS_PER_TILE,), jnp.float32),
        pltpu.VMEM_SHARED((N_PAD,), jnp.float32),
    ],
    compiler_params=_sc_params,
)
def _deg_dinv_kernel(dst_hbm, zeros_hbm, dinv_hbm, idx_v, ones_v, buf_v, degacc):
    c = lax.axis_index("c")
    s = lax.axis_index("s")

    @pl.when(c == 0)
    def _():
        base = s * ROWS_PER_TILE
        pltpu.sync_copy(
            zeros_hbm.at[pl.ds(base, ROWS_PER_TILE)],
            degacc.at[pl.ds(base, ROWS_PER_TILE)],
        )
        pltpu.sync_copy(dst_hbm.at[s], idx_v)

        @pl.loop(0, BLK // 16)
        def _(i):
            ones_v[pl.ds(i * 16, 16)] = jnp.ones((16,), jnp.float32)

        plsc.subcore_barrier()

        @pl.loop(0, NBLK_S)
        def _(j):
            pltpu.sync_copy(ones_v, degacc.at[idx_v.at[j]], add=True)

        plsc.subcore_barrier()

        pltpu.sync_copy(degacc.at[pl.ds(base, ROWS_PER_TILE)], buf_v)

        @pl.loop(0, ROWS_PER_TILE // 16)
        def _(i):
            v = buf_v[pl.ds(i * 16, 16)] + 1.0
            iv = lax.bitcast_convert_type(v, jnp.int32)
            iv = jnp.int32(0x5F3759DF) - (iv >> 1)
            y = lax.bitcast_convert_type(iv, jnp.float32)
            y = y * (1.5 - 0.5 * v * y * y)
            y = y * (1.5 - 0.5 * v * y * y)
            y = y * (1.5 - 0.5 * v * y * y)
            buf_v[pl.ds(i * 16, 16)] = y

        pltpu.sync_copy(buf_v, dinv_hbm.at[pl.ds(base, ROWS_PER_TILE)])


def _make_agg_kernel(feat):
    @functools.partial(
        pl.kernel,
        out_type=jax.ShapeDtypeStruct((NC, N_PAD, feat), jnp.float32),
        mesh=_mesh,
        scratch_types=[
            pltpu.VMEM((NBLK_W, BLK), jnp.int32),
            pltpu.VMEM((NBLK_W, BLK), jnp.int32),
            pltpu.VMEM((2, BLK, feat), jnp.float32),
            pltpu.VMEM_SHARED((N_PAD, feat), jnp.float32),
            pltpu.SemaphoreType.DMA,
        ],
        compiler_params=_sc_params,
    )
    def _agg(g_hbm, src_hbm, dst_hbm, zeros_hbm, out_hbm, srcv, dstv, rows, acc, sem):
        c = lax.axis_index("c")
        s = lax.axis_index("s")
        w = c * NS + s
        base = s * ROWS_PER_TILE
        pltpu.sync_copy(
            zeros_hbm.at[pl.ds(base, ROWS_PER_TILE)],
            acc.at[pl.ds(base, ROWS_PER_TILE)],
        )
        pltpu.sync_copy(src_hbm.at[w], srcv)
        pltpu.sync_copy(dst_hbm.at[w], dstv)
        plsc.subcore_barrier()

        @pl.loop(0, NBLK_W, step=2)
        def _(j):
            for b in range(2):
                pltpu.async_copy(g_hbm.at[srcv.at[j + b]], rows.at[b], sem).wait()
                pltpu.sync_copy(rows.at[b], acc.at[dstv.at[j + b]], add=True)

        plsc.subcore_barrier()
        pltpu.sync_copy(
            acc.at[pl.ds(base, ROWS_PER_TILE)],
            out_hbm.at[c, pl.ds(base, ROWS_PER_TILE)],
        )

    return _agg


_agg32 = _make_agg_kernel(32)
_agg16 = _make_agg_kernel(16)


def _tc1_body(x_ref, w1_ref, dinv_ref, g1_ref):
    h = jnp.dot(x_ref[...], w1_ref[...], preferred_element_type=jnp.float32)
    g1_ref[...] = h * dinv_ref[...]


def _tc2_body(agg_ref, g1_ref, dinv_ref, w2_ref, b1_ref, g2_ref):
    a = agg_ref[0] + agg_ref[1] + g1_ref[...]
    out1 = jnp.maximum(a * dinv_ref[...] + b1_ref[...], 0.0)
    h2 = jnp.dot(out1, w2_ref[...], preferred_element_type=jnp.float32)
    g2_ref[...] = h2 * dinv_ref[...]


def _tc3_body(agg_ref, g2_ref, dinv_ref, wl_ref, b2_ref, bl_ref, out_ref):
    a = agg_ref[0] + agg_ref[1] + g2_ref[...]
    out2 = jnp.maximum(a * dinv_ref[...] + b2_ref[...], 0.0)
    out_ref[...] = (
        jnp.dot(out2, wl_ref[...], preferred_element_type=jnp.float32) + bl_ref[...]
    )


def kernel(x, edge_index, W1, b1, W2, b2, Wl, bl):
    f32 = jnp.float32
    ei = edge_index.astype(jnp.int32)
    pad_e = E_PAD - N_EDGES
    src = jnp.concatenate([ei[0], jnp.zeros((pad_e,), jnp.int32)])
    dst = jnp.concatenate([ei[1], jnp.full((pad_e,), N_NODES, jnp.int32)])
    src3 = src.reshape(NC * NS, NBLK_W, BLK)
    dst3 = dst.reshape(NC * NS, NBLK_W, BLK)
    dst3_deg = dst.reshape(NS, NBLK_S, BLK)
    xp = jnp.pad(x, ((0, N_PAD - N_NODES), (0, 0)))
    zeros1 = jnp.zeros((N_PAD,), f32)
    zeros32 = jnp.zeros((N_PAD, 32), f32)
    zeros16 = jnp.zeros((N_PAD, 16), f32)

    dinv = _deg_dinv_kernel(dst3_deg, zeros1)
    dinv2d = dinv.reshape(N_PAD, 1)

    g1 = pl.pallas_call(
        _tc1_body, out_shape=jax.ShapeDtypeStruct((N_PAD, 32), f32)
    )(xp, W1, dinv2d)

    agg1 = _agg32(g1, src3, dst3, zeros32)

    g2 = pl.pallas_call(
        _tc2_body, out_shape=jax.ShapeDtypeStruct((N_PAD, 16), f32)
    )(agg1, g1, dinv2d, W2, b1.reshape(1, 32))

    agg2 = _agg16(g2, src3, dst3, zeros16)

    out = pl.pallas_call(
        _tc3_body, out_shape=jax.ShapeDtypeStruct((N_PAD, 1), f32)
    )(agg2, g2, dinv2d, Wl, b2.reshape(1, 16), bl.reshape(1, 1))

    return out[:N_NODES]


# speedup vs baseline: 30.2373x; 1.1825x over previous
import functools

import jax
import jax.numpy as jnp
from jax import lax
from jax.experimental import pallas as pl
from jax.experimental.pallas import tpu as pltpu
from jax.experimental.pallas import tpu_sc as plsc

N_NODES = 10000
N_PAD = 10240
N_EDGES = 320000
E_PAD = 327680
BLK = 128
NC = 2
NS = 16
ROWS_PER_TILE = N_PAD // NS
NBLK_W = E_PAD // (NC * NS * BLK)
NBLK_S = E_PAD // (NS * BLK)

_mesh = plsc.VectorSubcoreMesh(
    core_axis_name="c", subcore_axis_name="s", num_cores=NC, num_subcores=NS
)
_sc_params = pltpu.CompilerParams(use_tc_tiling_on_sc=False)


@functools.partial(
    pl.kernel,
    out_type=jax.ShapeDtypeStruct((N_PAD,), jnp.float32),
    mesh=_mesh,
    scratch_types=[
        pltpu.VMEM((NBLK_S, BLK), jnp.int32),
        pltpu.VMEM((BLK,), jnp.float32),
        pltpu.VMEM((ROWS_PER_TILE,), jnp.float32),
        pltpu.VMEM_SHARED((N_PAD,), jnp.float32),
        pltpu.SemaphoreType.DMA,
    ],
    compiler_params=_sc_params,
)
def _deg_dinv_kernel(dst_hbm, zeros_hbm, dinv_hbm, idx_v, ones_v, buf_v, degacc, sem):
    c = lax.axis_index("c")
    s = lax.axis_index("s")

    @pl.when(c == 0)
    def _():
        base = s * ROWS_PER_TILE
        pltpu.sync_copy(
            zeros_hbm.at[pl.ds(base, ROWS_PER_TILE)],
            degacc.at[pl.ds(base, ROWS_PER_TILE)],
        )
        pltpu.sync_copy(dst_hbm.at[s], idx_v)

        @pl.loop(0, BLK // 16)
        def _(i):
            ones_v[pl.ds(i * 16, 16)] = jnp.ones((16,), jnp.float32)

        plsc.subcore_barrier()

        @pl.loop(0, NBLK_S)
        def _(j):
            pltpu.async_copy(ones_v, degacc.at[idx_v.at[j]], sem, add=True)

        @pl.loop(0, NBLK_S)
        def _(j):
            pltpu.make_async_copy(ones_v, degacc.at[idx_v.at[j]], sem).wait()

        plsc.subcore_barrier()

        pltpu.sync_copy(degacc.at[pl.ds(base, ROWS_PER_TILE)], buf_v)

        @pl.loop(0, ROWS_PER_TILE // 16)
        def _(i):
            v = buf_v[pl.ds(i * 16, 16)] + 1.0
            iv = lax.bitcast_convert_type(v, jnp.int32)
            iv = jnp.int32(0x5F3759DF) - (iv >> 1)
            y = lax.bitcast_convert_type(iv, jnp.float32)
            y = y * (1.5 - 0.5 * v * y * y)
            y = y * (1.5 - 0.5 * v * y * y)
            y = y * (1.5 - 0.5 * v * y * y)
            buf_v[pl.ds(i * 16, 16)] = y

        pltpu.sync_copy(buf_v, dinv_hbm.at[pl.ds(base, ROWS_PER_TILE)])


def _make_agg_kernel(feat):
    @functools.partial(
        pl.kernel,
        out_type=jax.ShapeDtypeStruct((NC, N_PAD, feat), jnp.float32),
        mesh=_mesh,
        scratch_types=[
            pltpu.VMEM((NBLK_W, BLK), jnp.int32),
            pltpu.VMEM((NBLK_W, BLK), jnp.int32),
            pltpu.VMEM((2, BLK, feat), jnp.float32),
            pltpu.VMEM_SHARED((N_PAD, feat), jnp.float32),
            pltpu.SemaphoreType.DMA,
            pltpu.SemaphoreType.DMA,
            pltpu.SemaphoreType.DMA,
            pltpu.SemaphoreType.DMA,
        ],
        compiler_params=_sc_params,
    )
    def _agg(g_hbm, src_hbm, dst_hbm, zeros_hbm, out_hbm, srcv, dstv, rows, acc,
             semg0, semg1, sems0, sems1):
        c = lax.axis_index("c")
        s = lax.axis_index("s")
        w = c * NS + s
        base = s * ROWS_PER_TILE
        pltpu.sync_copy(
            zeros_hbm.at[pl.ds(base, ROWS_PER_TILE)],
            acc.at[pl.ds(base, ROWS_PER_TILE)],
        )
        pltpu.sync_copy(src_hbm.at[w], srcv)
        pltpu.sync_copy(dst_hbm.at[w], dstv)
        plsc.subcore_barrier()

        pltpu.async_copy(g_hbm.at[srcv.at[0]], rows.at[0], semg0)
        pltpu.async_copy(g_hbm.at[srcv.at[1]], rows.at[1], semg1)

        @pl.loop(0, NBLK_W, step=2)
        def _(j):
            pltpu.make_async_copy(g_hbm.at[srcv.at[j]], rows.at[0], semg0).wait()
            pltpu.async_copy(rows.at[0], acc.at[dstv.at[j]], sems0, add=True)
            pltpu.make_async_copy(g_hbm.at[srcv.at[j + 1]], rows.at[1], semg1).wait()
            pltpu.async_copy(rows.at[1], acc.at[dstv.at[j + 1]], sems1, add=True)
            pltpu.make_async_copy(rows.at[0], acc.at[dstv.at[j]], sems0).wait()

            @pl.when(j + 2 < NBLK_W)
            def _():
                pltpu.async_copy(g_hbm.at[srcv.at[j + 2]], rows.at[0], semg0)

            pltpu.make_async_copy(rows.at[1], acc.at[dstv.at[j + 1]], sems1).wait()

            @pl.when(j + 3 < NBLK_W)
            def _():
                pltpu.async_copy(g_hbm.at[srcv.at[j + 3]], rows.at[1], semg1)

        plsc.subcore_barrier()
        pltpu.sync_copy(
            acc.at[pl.ds(base, ROWS_PER_TILE)],
            out_hbm.at[c, pl.ds(base, ROWS_PER_TILE)],
        )

    return _agg


_agg32 = _make_agg_kernel(32)
_agg16 = _make_agg_kernel(16)


def _tc1_body(x_ref, w1_ref, dinv_ref, g1_ref):
    h = jnp.dot(x_ref[...], w1_ref[...], preferred_element_type=jnp.float32)
    g1_ref[...] = h * dinv_ref[...]


def _tc2_body(agg_ref, g1_ref, dinv_ref, w2_ref, b1_ref, g2_ref):
    a = agg_ref[0] + agg_ref[1] + g1_ref[...]
    out1 = jnp.maximum(a * dinv_ref[...] + b1_ref[...], 0.0)
    h2 = jnp.dot(out1, w2_ref[...], preferred_element_type=jnp.float32)
    g2_ref[...] = h2 * dinv_ref[...]


def _tc3_body(agg_ref, g2_ref, dinv_ref, wl_ref, b2_ref, bl_ref, out_ref):
    a = agg_ref[0] + agg_ref[1] + g2_ref[...]
    out2 = jnp.maximum(a * dinv_ref[...] + b2_ref[...], 0.0)
    out_ref[...] = (
        jnp.dot(out2, wl_ref[...], preferred_element_type=jnp.float32) + bl_ref[...]
    )


def kernel(x, edge_index, W1, b1, W2, b2, Wl, bl):
    f32 = jnp.float32
    ei = edge_index.astype(jnp.int32)
    pad_e = E_PAD - N_EDGES
    src = jnp.concatenate([ei[0], jnp.zeros((pad_e,), jnp.int32)])
    dst = jnp.concatenate([ei[1], jnp.full((pad_e,), N_NODES, jnp.int32)])
    src3 = src.reshape(NC * NS, NBLK_W, BLK)
    dst3 = dst.reshape(NC * NS, NBLK_W, BLK)
    dst3_deg = dst.reshape(NS, NBLK_S, BLK)
    xp = jnp.pad(x, ((0, N_PAD - N_NODES), (0, 0)))
    zeros1 = jnp.zeros((N_PAD,), f32)
    zeros32 = jnp.zeros((N_PAD, 32), f32)
    zeros16 = jnp.zeros((N_PAD, 16), f32)

    dinv = _deg_dinv_kernel(dst3_deg, zeros1)
    dinv2d = dinv.reshape(N_PAD, 1)

    g1 = pl.pallas_call(
        _tc1_body, out_shape=jax.ShapeDtypeStruct((N_PAD, 32), f32)
    )(xp, W1, dinv2d)

    agg1 = _agg32(g1, src3, dst3, zeros32)

    g2 = pl.pallas_call(
        _tc2_body, out_shape=jax.ShapeDtypeStruct((N_PAD, 16), f32)
    )(agg1, g1, dinv2d, W2, b1.reshape(1, 32))

    agg2 = _agg16(g2, src3, dst3, zeros16)

    out = pl.pallas_call(
        _tc3_body, out_shape=jax.ShapeDtypeStruct((N_PAD, 1), f32)
    )(agg2, g2, dinv2d, Wl, b2.reshape(1, 16), bl.reshape(1, 1))

    return out[:N_NODES]


# speedup vs baseline: 38.2157x; 1.2639x over previous
import functools

import jax
import jax.numpy as jnp
from jax import lax
from jax.experimental import pallas as pl
from jax.experimental.pallas import tpu as pltpu
from jax.experimental.pallas import tpu_sc as plsc

N_NODES = 10000
N_PAD = 10240
N_EDGES = 320000
E_PAD = 327680
BLK = 128
NC = 2
NS = 16
ROWS_PER_TILE = N_PAD // NS
NBLK_W = E_PAD // (NC * NS * BLK)
NBLK_S = E_PAD // (NS * BLK)

_mesh = plsc.VectorSubcoreMesh(
    core_axis_name="c", subcore_axis_name="s", num_cores=NC, num_subcores=NS
)
_sc_params = pltpu.CompilerParams(use_tc_tiling_on_sc=False)


@functools.partial(
    pl.kernel,
    out_type=jax.ShapeDtypeStruct((N_PAD,), jnp.float32),
    mesh=_mesh,
    scratch_types=[
        pltpu.VMEM((NBLK_S, BLK), jnp.int32),
        pltpu.VMEM((BLK,), jnp.float32),
        pltpu.VMEM((ROWS_PER_TILE,), jnp.float32),
        pltpu.VMEM_SHARED((N_PAD,), jnp.float32),
        pltpu.SemaphoreType.DMA,
    ],
    compiler_params=_sc_params,
)
def _deg_dinv_kernel(dst_hbm, zeros_hbm, dinv_hbm, idx_v, ones_v, buf_v, degacc, sem):
    c = lax.axis_index("c")
    s = lax.axis_index("s")

    @pl.when(c == 0)
    def _():
        base = s * ROWS_PER_TILE
        pltpu.sync_copy(
            zeros_hbm.at[pl.ds(base, ROWS_PER_TILE)],
            degacc.at[pl.ds(base, ROWS_PER_TILE)],
        )
        pltpu.sync_copy(dst_hbm.at[s], idx_v)

        @pl.loop(0, BLK // 16)
        def _(i):
            ones_v[pl.ds(i * 16, 16)] = jnp.ones((16,), jnp.float32)

        plsc.subcore_barrier()

        @pl.loop(0, NBLK_S)
        def _(j):
            pltpu.async_copy(ones_v, degacc.at[idx_v.at[j]], sem, add=True)

        @pl.loop(0, NBLK_S)
        def _(j):
            pltpu.make_async_copy(ones_v, degacc.at[idx_v.at[j]], sem).wait()

        plsc.subcore_barrier()

        pltpu.sync_copy(degacc.at[pl.ds(base, ROWS_PER_TILE)], buf_v)

        @pl.loop(0, ROWS_PER_TILE // 16)
        def _(i):
            v = buf_v[pl.ds(i * 16, 16)] + 1.0
            iv = lax.bitcast_convert_type(v, jnp.int32)
            iv = jnp.int32(0x5F3759DF) - (iv >> 1)
            y = lax.bitcast_convert_type(iv, jnp.float32)
            y = y * (1.5 - 0.5 * v * y * y)
            y = y * (1.5 - 0.5 * v * y * y)
            y = y * (1.5 - 0.5 * v * y * y)
            buf_v[pl.ds(i * 16, 16)] = y

        pltpu.sync_copy(buf_v, dinv_hbm.at[pl.ds(base, ROWS_PER_TILE)])


def _make_agg_kernel(feat):
    @functools.partial(
        pl.kernel,
        out_type=jax.ShapeDtypeStruct((NC, N_PAD, feat), jnp.bfloat16),
        mesh=_mesh,
        scratch_types=[
            pltpu.VMEM((NBLK_W, BLK), jnp.int32),
            pltpu.VMEM((NBLK_W, BLK), jnp.int32),
            pltpu.VMEM((2, BLK, feat), jnp.bfloat16),
            pltpu.VMEM_SHARED((N_PAD, feat), jnp.bfloat16),
            pltpu.SemaphoreType.DMA,
            pltpu.SemaphoreType.DMA,
            pltpu.SemaphoreType.DMA,
            pltpu.SemaphoreType.DMA,
        ],
        compiler_params=_sc_params,
    )
    def _agg(g_hbm, src_hbm, dst_hbm, zeros_hbm, out_hbm, srcv, dstv, rows, acc,
             semg0, semg1, sems0, sems1):
        c = lax.axis_index("c")
        s = lax.axis_index("s")
        w = c * NS + s
        base = s * ROWS_PER_TILE
        pltpu.sync_copy(
            zeros_hbm.at[pl.ds(base, ROWS_PER_TILE)],
            acc.at[pl.ds(base, ROWS_PER_TILE)],
        )
        pltpu.sync_copy(src_hbm.at[w], srcv)
        pltpu.sync_copy(dst_hbm.at[w], dstv)
        plsc.subcore_barrier()

        pltpu.async_copy(g_hbm.at[srcv.at[0]], rows.at[0], semg0)
        pltpu.async_copy(g_hbm.at[srcv.at[1]], rows.at[1], semg1)

        @pl.loop(0, NBLK_W, step=2)
        def _(j):
            pltpu.make_async_copy(g_hbm.at[srcv.at[j]], rows.at[0], semg0).wait()
            pltpu.async_copy(rows.at[0], acc.at[dstv.at[j]], sems0, add=True)
            pltpu.make_async_copy(g_hbm.at[srcv.at[j + 1]], rows.at[1], semg1).wait()
            pltpu.async_copy(rows.at[1], acc.at[dstv.at[j + 1]], sems1, add=True)
            pltpu.make_async_copy(rows.at[0], acc.at[dstv.at[j]], sems0).wait()

            @pl.when(j + 2 < NBLK_W)
            def _():
                pltpu.async_copy(g_hbm.at[srcv.at[j + 2]], rows.at[0], semg0)

            pltpu.make_async_copy(rows.at[1], acc.at[dstv.at[j + 1]], sems1).wait()

            @pl.when(j + 3 < NBLK_W)
            def _():
                pltpu.async_copy(g_hbm.at[srcv.at[j + 3]], rows.at[1], semg1)

        plsc.subcore_barrier()
        pltpu.sync_copy(
            acc.at[pl.ds(base, ROWS_PER_TILE)],
            out_hbm.at[c, pl.ds(base, ROWS_PER_TILE)],
        )

    return _agg


_agg32 = _make_agg_kernel(32)
_agg16 = _make_agg_kernel(16)


def _tc1_body(x_ref, w1_ref, dinv_ref, g1_ref):
    h = jnp.dot(x_ref[...], w1_ref[...], preferred_element_type=jnp.float32)
    g1_ref[...] = (h * dinv_ref[...]).astype(jnp.bfloat16)


def _tc2_body(agg_ref, g1_ref, dinv_ref, w2_ref, b1_ref, g2_ref):
    a = (agg_ref[0].astype(jnp.float32) + agg_ref[1].astype(jnp.float32)
         + g1_ref[...].astype(jnp.float32))
    out1 = jnp.maximum(a * dinv_ref[...] + b1_ref[...], 0.0)
    h2 = jnp.dot(out1, w2_ref[...], preferred_element_type=jnp.float32)
    g2_ref[...] = (h2 * dinv_ref[...]).astype(jnp.bfloat16)


def _tc3_body(agg_ref, g2_ref, dinv_ref, wl_ref, b2_ref, bl_ref, out_ref):
    a = (agg_ref[0].astype(jnp.float32) + agg_ref[1].astype(jnp.float32)
         + g2_ref[...].astype(jnp.float32))
    out2 = jnp.maximum(a * dinv_ref[...] + b2_ref[...], 0.0)
    out_ref[...] = (
        jnp.dot(out2, wl_ref[...], preferred_element_type=jnp.float32) + bl_ref[...]
    )


def kernel(x, edge_index, W1, b1, W2, b2, Wl, bl):
    f32 = jnp.float32
    ei = edge_index.astype(jnp.int32)
    pad_e = E_PAD - N_EDGES
    src = jnp.concatenate([ei[0], jnp.zeros((pad_e,), jnp.int32)])
    dst = jnp.concatenate([ei[1], jnp.full((pad_e,), N_NODES, jnp.int32)])
    src3 = src.reshape(NC * NS, NBLK_W, BLK)
    dst3 = dst.reshape(NC * NS, NBLK_W, BLK)
    dst3_deg = dst.reshape(NS, NBLK_S, BLK)
    xp = jnp.pad(x, ((0, N_PAD - N_NODES), (0, 0)))
    zeros1 = jnp.zeros((N_PAD,), f32)
    zeros32 = jnp.zeros((N_PAD, 32), jnp.bfloat16)
    zeros16 = jnp.zeros((N_PAD, 16), jnp.bfloat16)

    dinv = _deg_dinv_kernel(dst3_deg, zeros1)
    dinv2d = dinv.reshape(N_PAD, 1)

    g1 = pl.pallas_call(
        _tc1_body, out_shape=jax.ShapeDtypeStruct((N_PAD, 32), jnp.bfloat16)
    )(xp, W1, dinv2d)

    agg1 = _agg32(g1, src3, dst3, zeros32)

    g2 = pl.pallas_call(
        _tc2_body, out_shape=jax.ShapeDtypeStruct((N_PAD, 16), jnp.bfloat16)
    )(agg1, g1, dinv2d, W2, b1.reshape(1, 32))

    agg2 = _agg16(g2, src3, dst3, zeros16)

    out = pl.pallas_call(
        _tc3_body, out_shape=jax.ShapeDtypeStruct((N_PAD, 1), f32)
    )(agg2, g2, dinv2d, Wl, b2.reshape(1, 16), bl.reshape(1, 1))

    return out[:N_NODES]
